# Initial kernel scaffold; baseline (speedup 1.0000x reference)
import jax
import jax.numpy as jnp
from jax.experimental import pallas as pl


def kernel(x, edge_index, W0, b0, W1, b1, W2, b2):
    raise NotImplementedError("write your pallas kernel here")


# speedup vs baseline: 19.7308x; 19.7308x over previous
import functools

import jax
import jax.numpy as jnp
from jax import lax
from jax.experimental import pallas as pl
from jax.experimental.pallas import tpu as pltpu
from jax.experimental.pallas import tpu_sc as plsc

N = 10000
NPAD = 10240
E = 320000
D_IN = 128
D_H = 64
D_OUT = 64
NC = 2
NS = 16
NT = NC * NS
EPT = E // NT
CB = 80
NCH = EPT // CB
DW = 16
RPT = NPAD // NS
BM = 2000

_mesh = plsc.VectorSubcoreMesh(core_axis_name="c", subcore_axis_name="s",
                               num_cores=NC, num_subcores=NS)
_sc_params = pltpu.CompilerParams(use_tc_tiling_on_sc=False)


def _deg_body(dst_hbm, ones_hbm, zeros_hbm, out_hbm, idx_v, ones_v, acc_sh, sem):
    c = lax.axis_index("c")
    s = lax.axis_index("s")
    wid = s * NC + c
    pltpu.sync_copy(zeros_hbm.at[pl.ds(s * RPT, RPT)], acc_sh.at[pl.ds(s * RPT, RPT)])
    pltpu.sync_copy(dst_hbm.at[wid], idx_v)
    pltpu.sync_copy(ones_hbm, ones_v)
    plsc.subcore_barrier()

    @pl.loop(0, NCH)
    def _(j):
        pltpu.sync_copy(ones_v, acc_sh.at[idx_v.at[j]], add=True)

    plsc.subcore_barrier()
    pltpu.sync_copy(acc_sh.at[pl.ds(s * RPT, RPT)],
                    out_hbm.at[c, pl.ds(s * RPT, RPT)])


def _sc_degree(dst3, ones, zeros16):
    f = pl.kernel(
        _deg_body,
        out_type=jax.ShapeDtypeStruct((NC, NPAD, DW), jnp.float32),
        mesh=_mesh,
        scratch_types=[
            pltpu.VMEM((NCH, CB), jnp.int32),
            pltpu.VMEM((CB, DW), jnp.float32),
            pltpu.VMEM_SHARED((NPAD, DW), jnp.float32),
            pltpu.SemaphoreType.DMA,
        ],
        compiler_params=_sc_params,
    )
    return f(dst3, ones, zeros16)


def _scat_body(src_hbm, dst_hbm, g_hbm, zeros_hbm, out_hbm,
               si_v, di_v, buf_v, acc_sh, sem):
    c = lax.axis_index("c")
    s = lax.axis_index("s")
    wid = s * NC + c
    pltpu.sync_copy(zeros_hbm.at[pl.ds(s * RPT, RPT)], acc_sh.at[pl.ds(s * RPT, RPT)])
    pltpu.sync_copy(src_hbm.at[wid], si_v)
    pltpu.sync_copy(dst_hbm.at[wid], di_v)
    plsc.subcore_barrier()

    @pl.loop(0, NCH)
    def _(j):
        pltpu.sync_copy(g_hbm.at[si_v.at[j]], buf_v)
        pltpu.sync_copy(buf_v, acc_sh.at[di_v.at[j]], add=True)

    plsc.subcore_barrier()
    pltpu.sync_copy(acc_sh.at[pl.ds(s * RPT, RPT)],
                    out_hbm.at[c, pl.ds(s * RPT, RPT)])


def _sc_aggregate(src3, dst3, g, zeros64):
    f = pl.kernel(
        _scat_body,
        out_type=jax.ShapeDtypeStruct((NC, NPAD, D_H), jnp.float32),
        mesh=_mesh,
        scratch_types=[
            pltpu.VMEM((NCH, CB), jnp.int32),
            pltpu.VMEM((NCH, CB), jnp.int32),
            pltpu.VMEM((CB, D_H), jnp.float32),
            pltpu.VMEM_SHARED((NPAD, D_H), jnp.float32),
            pltpu.SemaphoreType.DMA,
        ],
        compiler_params=_sc_params,
    )
    return f(src3, dst3, g, zeros64)


def _dis_block(dga_ref, dgb_ref):
    deg = jnp.sum(dga_ref[...] + dgb_ref[...], axis=1, keepdims=True) * (1.0 / DW) + 1.0
    return 1.0 / jnp.sqrt(deg)


def _pre_body(x_ref, w_ref, dga_ref, dgb_ref, g_ref):
    dis = _dis_block(dga_ref, dgb_ref)
    h = jnp.dot(x_ref[...], w_ref[...], preferred_element_type=jnp.float32)
    g_ref[...] = h * dis


def _mid_body(aa_ref, ab_ref, g_ref, w_ref, b_ref, dga_ref, dgb_ref, o_ref):
    dis = _dis_block(dga_ref, dgb_ref)
    act = dis * (aa_ref[...] + ab_ref[...] + g_ref[...]) + b_ref[...]
    act = jnp.maximum(act, 0.0)
    h = jnp.dot(act, w_ref[...], preferred_element_type=jnp.float32)
    o_ref[...] = h * dis


def _fin_body(aa_ref, ab_ref, g_ref, b_ref, dga_ref, dgb_ref, o_ref):
    dis = _dis_block(dga_ref, dgb_ref)
    o_ref[...] = dis * (aa_ref[...] + ab_ref[...] + g_ref[...]) + b_ref[...]


def _row_spec(w):
    return pl.BlockSpec((BM, w), lambda i: (i, 0))


def _full_spec(h, w):
    return pl.BlockSpec((h, w), lambda i: (0, 0))


def _tc_pre(x, w0, dga, dgb):
    return pl.pallas_call(
        _pre_body,
        grid=(N // BM,),
        in_specs=[_row_spec(D_IN), _full_spec(D_IN, D_H), _row_spec(DW), _row_spec(DW)],
        out_specs=_row_spec(D_H),
        out_shape=jax.ShapeDtypeStruct((N, D_H), jnp.float32),
    )(x, w0, dga, dgb)


def _tc_mid(aa, ab, g, w, b, dga, dgb):
    return pl.pallas_call(
        _mid_body,
        grid=(N // BM,),
        in_specs=[_row_spec(D_H), _row_spec(D_H), _row_spec(D_H),
                  _full_spec(D_H, D_H), _full_spec(1, D_H),
                  _row_spec(DW), _row_spec(DW)],
        out_specs=_row_spec(D_H),
        out_shape=jax.ShapeDtypeStruct((N, D_H), jnp.float32),
    )(aa, ab, g, w, b, dga, dgb)


def _tc_fin(aa, ab, g, b, dga, dgb):
    return pl.pallas_call(
        _fin_body,
        grid=(N // BM,),
        in_specs=[_row_spec(D_H), _row_spec(D_H), _row_spec(D_H),
                  _full_spec(1, D_OUT), _row_spec(DW), _row_spec(DW)],
        out_specs=_row_spec(D_OUT),
        out_shape=jax.ShapeDtypeStruct((N, D_OUT), jnp.float32),
    )(aa, ab, g, b, dga, dgb)


def kernel(x, edge_index, W0, b0, W1, b1, W2, b2):
    ei = edge_index.astype(jnp.int32)
    src3 = ei[0].reshape(NT, NCH, CB)
    dst3 = ei[1].reshape(NT, NCH, CB)
    ones = jnp.ones((CB, DW), jnp.float32)
    zeros16 = jnp.zeros((NPAD, DW), jnp.float32)
    zeros64 = jnp.zeros((NPAD, D_H), jnp.float32)

    degp = _sc_degree(dst3, ones, zeros16)
    dga, dgb = degp[0], degp[1]

    g0 = _tc_pre(x, W0, dga, dgb)
    a0 = _sc_aggregate(src3, dst3, g0, zeros64)
    g1 = _tc_mid(a0[0, :N], a0[1, :N], g0, W1, b0.reshape(1, D_H), dga, dgb)
    a1 = _sc_aggregate(src3, dst3, g1, zeros64)
    g2 = _tc_mid(a1[0, :N], a1[1, :N], g1, W2, b1.reshape(1, D_H), dga, dgb)
    a2 = _sc_aggregate(src3, dst3, g2, zeros64)
    out = _tc_fin(a2[0, :N], a2[1, :N], g2, b2.reshape(1, D_OUT), dga, dgb)
    return out


# speedup vs baseline: 28.5231x; 1.4456x over previous
import functools

import jax
import jax.numpy as jnp
from jax import lax
from jax.experimental import pallas as pl
from jax.experimental.pallas import tpu as pltpu
from jax.experimental.pallas import tpu_sc as plsc

N = 10000
NPAD = 10240
E = 320000
D_IN = 128
D_H = 64
D_OUT = 64
NC = 2
NS = 16
NT = NC * NS
EPT = E // NT
CB = 80
NCH = EPT // CB
DW = 16
RPT = NPAD // NS
BM = 2000

_mesh = plsc.VectorSubcoreMesh(core_axis_name="c", subcore_axis_name="s",
                               num_cores=NC, num_subcores=NS)
_sc_params = pltpu.CompilerParams(use_tc_tiling_on_sc=False)


def _deg_body(dst_hbm, ones_hbm, zeros_hbm, out_hbm, idx_v, ones_v, acc_sh, sem):
    c = lax.axis_index("c")
    s = lax.axis_index("s")
    wid = s * NC + c
    pltpu.sync_copy(zeros_hbm.at[pl.ds(s * RPT, RPT)], acc_sh.at[pl.ds(s * RPT, RPT)])
    pltpu.sync_copy(dst_hbm.at[wid], idx_v)
    pltpu.sync_copy(ones_hbm, ones_v)
    plsc.subcore_barrier()

    @pl.loop(0, NCH)
    def _(j):
        pltpu.sync_copy(ones_v, acc_sh.at[idx_v.at[j]], add=True)

    plsc.subcore_barrier()
    pltpu.sync_copy(acc_sh.at[pl.ds(s * RPT, RPT)],
                    out_hbm.at[c, pl.ds(s * RPT, RPT)])


def _sc_degree(dst3, ones, zeros16):
    f = pl.kernel(
        _deg_body,
        out_type=jax.ShapeDtypeStruct((NC, NPAD, DW), jnp.float32),
        mesh=_mesh,
        scratch_types=[
            pltpu.VMEM((NCH, CB), jnp.int32),
            pltpu.VMEM((CB, DW), jnp.float32),
            pltpu.VMEM_SHARED((NPAD, DW), jnp.float32),
            pltpu.SemaphoreType.DMA,
        ],
        compiler_params=_sc_params,
    )
    return f(dst3, ones, zeros16)


def _scat_body(src_hbm, dst_hbm, g_hbm, zeros_hbm, out_hbm,
               si_v, di_v, bufa_v, bufb_v, acc_sh, sema, semb):
    c = lax.axis_index("c")
    s = lax.axis_index("s")
    wid = s * NC + c
    pltpu.sync_copy(zeros_hbm.at[pl.ds(s * RPT, RPT)], acc_sh.at[pl.ds(s * RPT, RPT)])
    pltpu.sync_copy(src_hbm.at[wid], si_v)
    pltpu.sync_copy(dst_hbm.at[wid], di_v)
    plsc.subcore_barrier()

    half = NCH // 2

    def _fire(j, buf, sem):
        return pltpu.async_copy(g_hbm.at[si_v.at[j]], buf, sem)

    _fire(0, bufa_v, sema)
    _fire(1, bufb_v, semb)

    @pl.loop(0, half)
    def _(t):
        ja = 2 * t
        pltpu.make_async_copy(g_hbm.at[si_v.at[ja]], bufa_v, sema).wait()
        pltpu.sync_copy(bufa_v, acc_sh.at[di_v.at[ja]], add=True)
        _fire(ja + 2, bufa_v, sema)

        jb = 2 * t + 1
        pltpu.make_async_copy(g_hbm.at[si_v.at[jb]], bufb_v, semb).wait()
        pltpu.sync_copy(bufb_v, acc_sh.at[di_v.at[jb]], add=True)

        @pl.when(t < half - 1)
        def _():
            _fire(jb + 2, bufb_v, semb)

    pltpu.make_async_copy(g_hbm.at[si_v.at[NCH - 1]], bufa_v, sema).wait()
    pltpu.sync_copy(bufa_v, acc_sh.at[di_v.at[NCH - 1]], add=True)

    plsc.subcore_barrier()
    pltpu.sync_copy(acc_sh.at[pl.ds(s * RPT, RPT)],
                    out_hbm.at[c, pl.ds(s * RPT, RPT)])


def _sc_aggregate(src3, dst3, g, zeros64):
    f = pl.kernel(
        _scat_body,
        out_type=jax.ShapeDtypeStruct((NC, NPAD, D_H), jnp.float32),
        mesh=_mesh,
        scratch_types=[
            pltpu.VMEM((NCH, CB), jnp.int32),
            pltpu.VMEM((NCH, CB), jnp.int32),
            pltpu.VMEM((CB, D_H), jnp.float32),
            pltpu.VMEM((CB, D_H), jnp.float32),
            pltpu.VMEM_SHARED((NPAD, D_H), jnp.float32),
            pltpu.SemaphoreType.DMA,
            pltpu.SemaphoreType.DMA,
        ],
        compiler_params=_sc_params,
    )
    return f(src3, dst3, g, zeros64)


def _dis_block(dga_ref, dgb_ref):
    deg = jnp.sum(dga_ref[...] + dgb_ref[...], axis=1, keepdims=True) * (1.0 / DW) + 1.0
    return 1.0 / jnp.sqrt(deg)


def _pre_body(x_ref, w_ref, dga_ref, dgb_ref, g_ref):
    dis = _dis_block(dga_ref, dgb_ref)
    h = jnp.dot(x_ref[...], w_ref[...], preferred_element_type=jnp.float32)
    g_ref[...] = h * dis


def _mid_body(aa_ref, ab_ref, g_ref, w_ref, b_ref, dga_ref, dgb_ref, o_ref):
    dis = _dis_block(dga_ref, dgb_ref)
    act = dis * (aa_ref[...] + ab_ref[...] + g_ref[...]) + b_ref[...]
    act = jnp.maximum(act, 0.0)
    h = jnp.dot(act, w_ref[...], preferred_element_type=jnp.float32)
    o_ref[...] = h * dis


def _fin_body(aa_ref, ab_ref, g_ref, b_ref, dga_ref, dgb_ref, o_ref):
    dis = _dis_block(dga_ref, dgb_ref)
    o_ref[...] = dis * (aa_ref[...] + ab_ref[...] + g_ref[...]) + b_ref[...]


def _row_spec(w):
    return pl.BlockSpec((BM, w), lambda i: (i, 0))


def _full_spec(h, w):
    return pl.BlockSpec((h, w), lambda i: (0, 0))


def _tc_pre(x, w0, dga, dgb):
    return pl.pallas_call(
        _pre_body,
        grid=(N // BM,),
        in_specs=[_row_spec(D_IN), _full_spec(D_IN, D_H), _row_spec(DW), _row_spec(DW)],
        out_specs=_row_spec(D_H),
        out_shape=jax.ShapeDtypeStruct((N, D_H), jnp.float32),
    )(x, w0, dga, dgb)


def _tc_mid(aa, ab, g, w, b, dga, dgb):
    return pl.pallas_call(
        _mid_body,
        grid=(N // BM,),
        in_specs=[_row_spec(D_H), _row_spec(D_H), _row_spec(D_H),
                  _full_spec(D_H, D_H), _full_spec(1, D_H),
                  _row_spec(DW), _row_spec(DW)],
        out_specs=_row_spec(D_H),
        out_shape=jax.ShapeDtypeStruct((N, D_H), jnp.float32),
    )(aa, ab, g, w, b, dga, dgb)


def _tc_fin(aa, ab, g, b, dga, dgb):
    return pl.pallas_call(
        _fin_body,
        grid=(N // BM,),
        in_specs=[_row_spec(D_H), _row_spec(D_H), _row_spec(D_H),
                  _full_spec(1, D_OUT), _row_spec(DW), _row_spec(DW)],
        out_specs=_row_spec(D_OUT),
        out_shape=jax.ShapeDtypeStruct((N, D_OUT), jnp.float32),
    )(aa, ab, g, b, dga, dgb)


def kernel(x, edge_index, W0, b0, W1, b1, W2, b2):
    ei = edge_index.astype(jnp.int32)
    src3 = ei[0].reshape(NT, NCH, CB)
    dst3 = ei[1].reshape(NT, NCH, CB)
    ones = jnp.ones((CB, DW), jnp.float32)
    zeros16 = jnp.zeros((NPAD, DW), jnp.float32)
    zeros64 = jnp.zeros((NPAD, D_H), jnp.float32)

    degp = _sc_degree(dst3, ones, zeros16)
    dga, dgb = degp[0], degp[1]

    g0 = _tc_pre(x, W0, dga, dgb)
    a0 = _sc_aggregate(src3, dst3, g0, zeros64)
    g1 = _tc_mid(a0[0, :N], a0[1, :N], g0, W1, b0.reshape(1, D_H), dga, dgb)
    a1 = _sc_aggregate(src3, dst3, g1, zeros64)
    g2 = _tc_mid(a1[0, :N], a1[1, :N], g1, W2, b1.reshape(1, D_H), dga, dgb)
    a2 = _sc_aggregate(src3, dst3, g2, zeros64)
    out = _tc_fin(a2[0, :N], a2[1, :N], g2, b2.reshape(1, D_OUT), dga, dgb)
    return out
